# Initial kernel scaffold; baseline (speedup 1.0000x reference)
import jax
import jax.numpy as jnp
from jax.experimental import pallas as pl


def kernel(bboxes, labels, prior_boxes):
    raise NotImplementedError("write your pallas kernel here")


# speedup vs baseline: 1.0485x; 1.0485x over previous
import functools

import jax
import jax.numpy as jnp
from jax import lax
from jax.experimental import pallas as pl
from jax.experimental.pallas import tpu as pltpu
from jax.experimental.pallas import tpu_sc as plsc

N = 5000
NPAD = 5120
P = 12240
PPAD = 12288
BT = 1024
PT = 512
NBT = NPAD // BT
NPT = PPAD // PT
NEG_T = 0.4
POS_T = 0.5
BIG = 2 ** 30


def _k1_body(bb_ref, pt_ref, match_ref, miou_ref, cx_ref, cy_ref, w_ref, h_ref,
             cmax_all, carg_all, rmax_s, rarg_s, win_s):
    ph = pl.program_id(0)
    p = pl.program_id(1)
    b = pl.program_id(2)

    bb = bb_ref[...]
    bx1 = bb[:, 0:1]
    by1 = bb[:, 1:2]
    bx2 = bb[:, 2:3]
    by2 = bb[:, 3:4]
    pt = pt_ref[...]
    pcx = pt[0:1, :]
    pcy = pt[1:2, :]
    pww = pt[2:3, :]
    phh = pt[3:4, :]

    cx_ref[0] = (bx1 + bx2) * 0.5
    cy_ref[0] = (by1 + by2) * 0.5
    w_ref[0] = bx2 - bx1
    h_ref[0] = by2 - by1

    rid = lax.broadcasted_iota(jnp.int32, (BT, PT), 0) + b * BT
    cid = lax.broadcasted_iota(jnp.int32, (BT, PT), 1) + p * PT

    @pl.when(ph == 0)
    def _phase0():
        px1 = pcx - pww * 0.5
        py1 = pcy - phh * 0.5
        px2 = pcx + pww * 0.5
        py2 = pcy + phh * 0.5
        area1 = (bx2 - bx1) * (by2 - by1)
        area2 = (px2 - px1) * (py2 - py1)
        ltx = jnp.maximum(bx1, px1)
        lty = jnp.maximum(by1, py1)
        rbx = jnp.minimum(bx2, px2)
        rby = jnp.minimum(by2, py2)
        whx = jnp.maximum(rbx - ltx, 0.0)
        why = jnp.maximum(rby - lty, 0.0)
        inter = whx * why
        union = (area1 + area2) - inter
        iou = inter / union

        cmax_t = jnp.max(iou, axis=0, keepdims=True)
        carg_t = jnp.min(jnp.where(iou == cmax_t, rid, BIG), axis=0,
                         keepdims=True)

        @pl.when(b == 0)
        def _():
            cmax_all[p] = cmax_t
            carg_all[p] = carg_t

        @pl.when(b > 0)
        def _():
            cur = cmax_all[p]
            cura = carg_all[p]
            upd = cmax_t > cur
            cmax_all[p] = jnp.where(upd, cmax_t, cur)
            carg_all[p] = jnp.where(upd, carg_t, cura)

        rmax_t = jnp.max(iou, axis=1, keepdims=True)
        rarg_t = jnp.min(jnp.where(iou == rmax_t, cid, BIG), axis=1,
                         keepdims=True)
        bsl = pl.ds(b * BT, BT)

        @pl.when(p == 0)
        def _():
            rmax_s[bsl, :] = rmax_t
            rarg_s[bsl, :] = rarg_t

        @pl.when(p > 0)
        def _():
            cur = rmax_s[bsl, :]
            cura = rarg_s[bsl, :]
            upd = rmax_t > cur
            rmax_s[bsl, :] = jnp.where(upd, rmax_t, cur)
            rarg_s[bsl, :] = jnp.where(upd, rarg_t, cura)

        match_ref[0] = carg_all[p]
        miou_ref[0] = cmax_all[p]

    @pl.when(ph == 1)
    def _phase1():
        fa = rarg_s[pl.ds(b * BT, BT), :]
        eq = (fa == cid) & (rid < N)
        cand = jnp.where(eq, rid, -1)
        wt = jnp.max(cand, axis=0, keepdims=True)

        @pl.when(b == 0)
        def _():
            win_s[...] = wt

        @pl.when(b > 0)
        def _():
            win_s[...] = jnp.maximum(win_s[...], wt)

        win = win_s[...]
        has = win >= 0
        match_ref[0] = jnp.where(has, win, carg_all[p])
        miou_ref[0] = jnp.where(has, POS_T, cmax_all[p])


def _k1_spec():
    return dict(
        grid=(2, NPT, NBT),
        in_specs=[
            pl.BlockSpec((BT, 4), lambda ph, p, b: (b, 0)),
            pl.BlockSpec((4, PT), lambda ph, p, b: (0, p)),
        ],
        out_specs=[
            pl.BlockSpec((1, 1, PT), lambda ph, p, b: (p, 0, 0)),
            pl.BlockSpec((1, 1, PT), lambda ph, p, b: (p, 0, 0)),
            pl.BlockSpec((1, BT, 1), lambda ph, p, b: (b, 0, 0)),
            pl.BlockSpec((1, BT, 1), lambda ph, p, b: (b, 0, 0)),
            pl.BlockSpec((1, BT, 1), lambda ph, p, b: (b, 0, 0)),
            pl.BlockSpec((1, BT, 1), lambda ph, p, b: (b, 0, 0)),
        ],
        out_shape=[
            jax.ShapeDtypeStruct((NPT, 1, PT), jnp.int32),
            jax.ShapeDtypeStruct((NPT, 1, PT), jnp.float32),
            jax.ShapeDtypeStruct((NBT, BT, 1), jnp.float32),
            jax.ShapeDtypeStruct((NBT, BT, 1), jnp.float32),
            jax.ShapeDtypeStruct((NBT, BT, 1), jnp.float32),
            jax.ShapeDtypeStruct((NBT, BT, 1), jnp.float32),
        ],
        scratch_shapes=[
            pltpu.VMEM((NPT, 1, PT), jnp.float32),
            pltpu.VMEM((NPT, 1, PT), jnp.int32),
            pltpu.VMEM((NPAD, 1), jnp.float32),
            pltpu.VMEM((NPAD, 1), jnp.int32),
            pltpu.VMEM((1, PT), jnp.int32),
        ],
    )


LN2 = 0.6931471805599453
SQRT2 = 1.4142135623730951


def _ln(x):
    bits = plsc.bitcast(x, jnp.int32)
    e = ((bits >> 23) & 0xFF) - 127
    m = plsc.bitcast((bits & 0x007FFFFF) | 0x3F800000, jnp.float32)
    big = m > SQRT2
    m = jnp.where(big, m * 0.5, m)
    e = jnp.where(big, e + 1, e)
    ef = e.astype(jnp.float32)
    z = (m - 1.0) / (m + 1.0)
    z2 = z * z
    poly = 2.0 + z2 * (2.0 / 3.0 + z2 * (2.0 / 5.0 + z2 * (2.0 / 7.0)))
    return ef * LN2 + z * poly


SLAB = PPAD // 32
NGV = SLAB // 16
NCV = NPAD // 16


def _k2_body(match_hbm, miou_hbm, cx_hbm, cy_hbm, w_hbm, h_hbm, lab_hbm,
             pcx_hbm, pcy_hbm, pw_hbm, ph_hbm,
             dcx_hbm, dcy_hbm, dw_hbm, dh_hbm, cls_hbm,
             cx_v, cy_v, w_v, h_v, lab_v,
             match_v, miou_v, pcx_v, pcy_v, pw_v, ph_v,
             odcx, odcy, odw, odh, ocls):
    wid = lax.axis_index("s") * 2 + lax.axis_index("c")
    base = wid * SLAB
    sl = pl.ds(base, SLAB)

    pltpu.sync_copy(cx_hbm, cx_v)
    pltpu.sync_copy(cy_hbm, cy_v)
    pltpu.sync_copy(w_hbm, w_v)
    pltpu.sync_copy(h_hbm, h_v)
    pltpu.sync_copy(lab_hbm, lab_v)
    pltpu.sync_copy(match_hbm.at[sl], match_v)
    pltpu.sync_copy(miou_hbm.at[sl], miou_v)
    pltpu.sync_copy(pcx_hbm.at[sl], pcx_v)
    pltpu.sync_copy(pcy_hbm.at[sl], pcy_v)
    pltpu.sync_copy(pw_hbm.at[sl], pw_v)
    pltpu.sync_copy(ph_hbm.at[sl], ph_v)

    def step(t, carry):
        s = pl.ds(t * 16, 16)
        idx = match_v[s]
        mcx = plsc.load_gather(cx_v, [idx])
        mcy = plsc.load_gather(cy_v, [idx])
        mw = plsc.load_gather(w_v, [idx])
        mh = plsc.load_gather(h_v, [idx])
        lab = plsc.load_gather(lab_v, [idx])
        pcx = pcx_v[s]
        pcy = pcy_v[s]
        pw = pw_v[s]
        phv = ph_v[s]
        miou = miou_v[s]
        odcx[s] = ((mcx - pcx) / pw) / 0.1
        odcy[s] = ((mcy - pcy) / phv) / 0.1
        odw[s] = _ln(mw / pw) / 0.2
        odh[s] = _ln(mh / phv) / 0.2
        cls = jnp.where(miou < POS_T, -1, lab)
        cls = jnp.where(miou < NEG_T, 0, cls)
        ocls[s] = cls
        return carry

    lax.fori_loop(0, NGV, step, 0)

    pltpu.sync_copy(odcx, dcx_hbm.at[sl])
    pltpu.sync_copy(odcy, dcy_hbm.at[sl])
    pltpu.sync_copy(odw, dw_hbm.at[sl])
    pltpu.sync_copy(odh, dh_hbm.at[sl])
    pltpu.sync_copy(ocls, cls_hbm.at[sl])


def _k2_call():
    f32 = jnp.float32
    i32 = jnp.int32
    return pl.kernel(
        _k2_body,
        out_type=[
            jax.ShapeDtypeStruct((PPAD,), f32),
            jax.ShapeDtypeStruct((PPAD,), f32),
            jax.ShapeDtypeStruct((PPAD,), f32),
            jax.ShapeDtypeStruct((PPAD,), f32),
            jax.ShapeDtypeStruct((PPAD,), i32),
        ],
        mesh=plsc.VectorSubcoreMesh(core_axis_name="c", subcore_axis_name="s"),
        compiler_params=pltpu.CompilerParams(needs_layout_passes=False),
        scratch_types=[
            pltpu.VMEM((NPAD,), f32),
            pltpu.VMEM((NPAD,), f32),
            pltpu.VMEM((NPAD,), f32),
            pltpu.VMEM((NPAD,), f32),
            pltpu.VMEM((NPAD,), i32),
            pltpu.VMEM((SLAB,), i32),
            pltpu.VMEM((SLAB,), f32),
            pltpu.VMEM((SLAB,), f32),
            pltpu.VMEM((SLAB,), f32),
            pltpu.VMEM((SLAB,), f32),
            pltpu.VMEM((SLAB,), f32),
            pltpu.VMEM((SLAB,), f32),
            pltpu.VMEM((SLAB,), f32),
            pltpu.VMEM((SLAB,), f32),
            pltpu.VMEM((SLAB,), f32),
            pltpu.VMEM((SLAB,), i32),
        ],
    )


def kernel(bboxes, labels, prior_boxes):
    f32 = jnp.float32
    i32 = jnp.int32
    pad_box = jnp.broadcast_to(
        jnp.array([-1e6, -1e6, -1e6 + 1.0, -1e6 + 1.0], f32), (NPAD - N, 4))
    bb_p = jnp.concatenate([bboxes.astype(f32), pad_box], axis=0)
    lab_p = jnp.concatenate([labels.astype(i32),
                             jnp.zeros((NPAD - N,), i32)], axis=0)
    pad_pri = jnp.broadcast_to(
        jnp.array([[-1e6], [-1e6], [1.0], [1.0]], f32), (4, PPAD - P))
    pri_t = jnp.concatenate([prior_boxes.astype(f32).T, pad_pri], axis=1)

    spec = _k1_spec()
    match3, miou3, cxc, cyc, wc, hc = pl.pallas_call(
        _k1_body,
        grid=spec["grid"],
        in_specs=spec["in_specs"],
        out_specs=spec["out_specs"],
        out_shape=spec["out_shape"],
        scratch_shapes=spec["scratch_shapes"],
        compiler_params=pltpu.CompilerParams(
            dimension_semantics=("arbitrary", "arbitrary", "arbitrary")),
    )(bb_p, pri_t)

    match = match3.reshape(PPAD)
    miou = miou3.reshape(PPAD)
    dcx, dcy, dw, dh, cls = _k2_call()(
        match, miou,
        cxc.reshape(NPAD), cyc.reshape(NPAD),
        wc.reshape(NPAD), hc.reshape(NPAD), lab_p,
        pri_t[0], pri_t[1], pri_t[2], pri_t[3])

    reg = jnp.stack([dcx[:P], dcy[:P], dw[:P], dh[:P]], axis=-1)
    return reg, cls[:P]


# speedup vs baseline: 1.5347x; 1.4636x over previous
import functools

import jax
import jax.numpy as jnp
from jax import lax
from jax.experimental import pallas as pl
from jax.experimental.pallas import tpu as pltpu
from jax.experimental.pallas import tpu_sc as plsc

N = 5000
NPAD = 5120
P = 12240
PPAD = 12288
BT = 1024
PT = 512
NBT = NPAD // BT
NPT = PPAD // PT
NEG_T = 0.4
POS_T = 0.5
BIG = 2 ** 30


def _k1_body(bb_ref, pt_ref, match_ref, miou_ref,
             cmax_all, carg_all, racc_v, racc_i, rarg_s, win_s):
    ph = pl.program_id(0)
    p = pl.program_id(1)
    b = pl.program_id(2)

    bb = bb_ref[...]
    bx1 = bb[:, 0:1]
    by1 = bb[:, 1:2]
    bx2 = bb[:, 2:3]
    by2 = bb[:, 3:4]
    pt = pt_ref[...]
    pcx = pt[0:1, :]
    pcy = pt[1:2, :]
    pww = pt[2:3, :]
    phh = pt[3:4, :]

    rid = lax.broadcasted_iota(jnp.int32, (BT, PT), 0) + b * BT
    bsl = pl.ds(b * BT, BT)

    @pl.when(ph == 0)
    def _phase0():
        px1 = pcx - pww * 0.5
        py1 = pcy - phh * 0.5
        px2 = pcx + pww * 0.5
        py2 = pcy + phh * 0.5
        area1 = (bx2 - bx1) * (by2 - by1)
        area2 = (px2 - px1) * (py2 - py1)
        ltx = jnp.maximum(bx1, px1)
        lty = jnp.maximum(by1, py1)
        rbx = jnp.minimum(bx2, px2)
        rby = jnp.minimum(by2, py2)
        whx = jnp.maximum(rbx - ltx, 0.0)
        why = jnp.maximum(rby - lty, 0.0)
        inter = whx * why
        union = (area1 + area2) - inter
        iou = inter / union

        cmax_t = jnp.max(iou, axis=0, keepdims=True)
        carg_t = jnp.min(jnp.where(iou == cmax_t, rid, BIG), axis=0,
                         keepdims=True)

        @pl.when(b == 0)
        def _():
            cmax_all[p] = cmax_t
            carg_all[p] = carg_t

        @pl.when(b > 0)
        def _():
            cur = cmax_all[p]
            cura = carg_all[p]
            upd = cmax_t > cur
            cmax_all[p] = jnp.where(upd, cmax_t, cur)
            carg_all[p] = jnp.where(upd, carg_t, cura)

        def fold(av, ai, g):
            v = iou[:, g * 128:(g + 1) * 128]
            gidx = (lax.broadcasted_iota(jnp.int32, (BT, 128), 1)
                    + (p * PT + g * 128))
            gt = v > av
            return jnp.where(gt, v, av), jnp.where(gt, gidx, ai)

        @pl.when(p == 0)
        def _():
            av = jnp.full((BT, 128), -1.0, jnp.float32)
            ai = jnp.full((BT, 128), BIG, jnp.int32)
            for g in range(4):
                av, ai = fold(av, ai, g)
            racc_v[bsl, :] = av
            racc_i[bsl, :] = ai

        @pl.when(p > 0)
        def _():
            av = racc_v[bsl, :]
            ai = racc_i[bsl, :]
            for g in range(4):
                av, ai = fold(av, ai, g)
            racc_v[bsl, :] = av
            racc_i[bsl, :] = ai

        match_ref[0] = carg_all[p]
        miou_ref[0] = cmax_all[p]

    @pl.when(ph == 1)
    def _phase1():
        cid = lax.broadcasted_iota(jnp.int32, (BT, PT), 1) + p * PT

        @pl.when(p == 0)
        def _():
            av = racc_v[bsl, :]
            ai = racc_i[bsl, :]
            rv = jnp.max(av, axis=1, keepdims=True)
            rarg_s[bsl, :] = jnp.min(jnp.where(av == rv, ai, BIG), axis=1,
                                     keepdims=True)

        fa = rarg_s[bsl, :]
        eq = (fa == cid) & (rid < N)
        cand = jnp.where(eq, rid, -1)
        wt = jnp.max(cand, axis=0, keepdims=True)

        @pl.when(b == 0)
        def _():
            win_s[...] = wt

        @pl.when(b > 0)
        def _():
            win_s[...] = jnp.maximum(win_s[...], wt)

        win = win_s[...]
        has = win >= 0
        match_ref[0] = jnp.where(has, win, carg_all[p])
        miou_ref[0] = jnp.where(has, POS_T, cmax_all[p])


def _k1_spec():
    return dict(
        grid=(2, NPT, NBT),
        in_specs=[
            pl.BlockSpec((BT, 4), lambda ph, p, b: (b, 0)),
            pl.BlockSpec((4, PT), lambda ph, p, b: (0, p)),
        ],
        out_specs=[
            pl.BlockSpec((1, 1, PT), lambda ph, p, b: (p, 0, 0)),
            pl.BlockSpec((1, 1, PT), lambda ph, p, b: (p, 0, 0)),
        ],
        out_shape=[
            jax.ShapeDtypeStruct((NPT, 1, PT), jnp.int32),
            jax.ShapeDtypeStruct((NPT, 1, PT), jnp.float32),
        ],
        scratch_shapes=[
            pltpu.VMEM((NPT, 1, PT), jnp.float32),
            pltpu.VMEM((NPT, 1, PT), jnp.int32),
            pltpu.VMEM((NPAD, 128), jnp.float32),
            pltpu.VMEM((NPAD, 128), jnp.int32),
            pltpu.VMEM((NPAD, 1), jnp.int32),
            pltpu.VMEM((1, PT), jnp.int32),
        ],
    )


LN2 = 0.6931471805599453
SQRT2 = 1.4142135623730951


def _ln(x):
    bits = plsc.bitcast(x, jnp.int32)
    e = ((bits >> 23) & 0xFF) - 127
    m = plsc.bitcast((bits & 0x007FFFFF) | 0x3F800000, jnp.float32)
    big = m > SQRT2
    m = jnp.where(big, m * 0.5, m)
    e = jnp.where(big, e + 1, e)
    ef = e.astype(jnp.float32)
    z = (m - 1.0) / (m + 1.0)
    z2 = z * z
    poly = 2.0 + z2 * (2.0 / 3.0 + z2 * (2.0 / 5.0 + z2 * (2.0 / 7.0)))
    return ef * LN2 + z * poly


SLAB = PPAD // 32
NGV = SLAB // 16
NCV = NPAD // 16


def _k2_body(match_hbm, miou_hbm, x1_hbm, y1_hbm, x2_hbm, y2_hbm, lab_hbm,
             pcx_hbm, pcy_hbm, pw_hbm, ph_hbm,
             dcx_hbm, dcy_hbm, dw_hbm, dh_hbm, cls_hbm,
             cx_v, cy_v, w_v, h_v, lab_v,
             match_v, miou_v, pcx_v, pcy_v, pw_v, ph_v,
             odcx, odcy, odw, odh, ocls):
    wid = lax.axis_index("s") * 2 + lax.axis_index("c")
    base = wid * SLAB
    sl = pl.ds(base, SLAB)

    pltpu.sync_copy(x1_hbm, cx_v)
    pltpu.sync_copy(x2_hbm, w_v)
    pltpu.sync_copy(y1_hbm, cy_v)
    pltpu.sync_copy(y2_hbm, h_v)
    pltpu.sync_copy(lab_hbm, lab_v)
    pltpu.sync_copy(match_hbm.at[sl], match_v)
    pltpu.sync_copy(miou_hbm.at[sl], miou_v)
    pltpu.sync_copy(pcx_hbm.at[sl], pcx_v)
    pltpu.sync_copy(pcy_hbm.at[sl], pcy_v)
    pltpu.sync_copy(pw_hbm.at[sl], pw_v)
    pltpu.sync_copy(ph_hbm.at[sl], ph_v)

    def cvt(t, carry):
        s = pl.ds(t * 16, 16)
        x1 = cx_v[s]
        x2 = w_v[s]
        y1 = cy_v[s]
        y2 = h_v[s]
        cx_v[s] = (x1 + x2) * 0.5
        w_v[s] = x2 - x1
        cy_v[s] = (y1 + y2) * 0.5
        h_v[s] = y2 - y1
        return carry

    lax.fori_loop(0, NCV, cvt, 0)

    def step(t, carry):
        s = pl.ds(t * 16, 16)
        idx = match_v[s]
        mcx = plsc.load_gather(cx_v, [idx])
        mcy = plsc.load_gather(cy_v, [idx])
        mw = plsc.load_gather(w_v, [idx])
        mh = plsc.load_gather(h_v, [idx])
        lab = plsc.load_gather(lab_v, [idx])
        pcx = pcx_v[s]
        pcy = pcy_v[s]
        pw = pw_v[s]
        phv = ph_v[s]
        miou = miou_v[s]
        odcx[s] = ((mcx - pcx) / pw) / 0.1
        odcy[s] = ((mcy - pcy) / phv) / 0.1
        odw[s] = _ln(mw / pw) / 0.2
        odh[s] = _ln(mh / phv) / 0.2
        cls = jnp.where(miou < POS_T, -1, lab)
        cls = jnp.where(miou < NEG_T, 0, cls)
        ocls[s] = cls
        return carry

    lax.fori_loop(0, NGV, step, 0)

    pltpu.sync_copy(odcx, dcx_hbm.at[sl])
    pltpu.sync_copy(odcy, dcy_hbm.at[sl])
    pltpu.sync_copy(odw, dw_hbm.at[sl])
    pltpu.sync_copy(odh, dh_hbm.at[sl])
    pltpu.sync_copy(ocls, cls_hbm.at[sl])


def _k2_call():
    f32 = jnp.float32
    i32 = jnp.int32
    return pl.kernel(
        _k2_body,
        out_type=[
            jax.ShapeDtypeStruct((PPAD,), f32),
            jax.ShapeDtypeStruct((PPAD,), f32),
            jax.ShapeDtypeStruct((PPAD,), f32),
            jax.ShapeDtypeStruct((PPAD,), f32),
            jax.ShapeDtypeStruct((PPAD,), i32),
        ],
        mesh=plsc.VectorSubcoreMesh(core_axis_name="c", subcore_axis_name="s"),
        compiler_params=pltpu.CompilerParams(needs_layout_passes=False),
        scratch_types=[
            pltpu.VMEM((NPAD,), f32),
            pltpu.VMEM((NPAD,), f32),
            pltpu.VMEM((NPAD,), f32),
            pltpu.VMEM((NPAD,), f32),
            pltpu.VMEM((NPAD,), i32),
            pltpu.VMEM((SLAB,), i32),
            pltpu.VMEM((SLAB,), f32),
            pltpu.VMEM((SLAB,), f32),
            pltpu.VMEM((SLAB,), f32),
            pltpu.VMEM((SLAB,), f32),
            pltpu.VMEM((SLAB,), f32),
            pltpu.VMEM((SLAB,), f32),
            pltpu.VMEM((SLAB,), f32),
            pltpu.VMEM((SLAB,), f32),
            pltpu.VMEM((SLAB,), f32),
            pltpu.VMEM((SLAB,), i32),
        ],
    )


def kernel(bboxes, labels, prior_boxes):
    f32 = jnp.float32
    i32 = jnp.int32
    pad_box = jnp.broadcast_to(
        jnp.array([-1e6, -1e6, -1e6 + 1.0, -1e6 + 1.0], f32), (NPAD - N, 4))
    bb_p = jnp.concatenate([bboxes.astype(f32), pad_box], axis=0)
    lab_p = jnp.concatenate([labels.astype(i32),
                             jnp.zeros((NPAD - N,), i32)], axis=0)
    pad_pri = jnp.broadcast_to(
        jnp.array([[-1e6], [-1e6], [1.0], [1.0]], f32), (4, PPAD - P))
    pri_t = jnp.concatenate([prior_boxes.astype(f32).T, pad_pri], axis=1)

    spec = _k1_spec()
    match3, miou3 = pl.pallas_call(
        _k1_body,
        grid=spec["grid"],
        in_specs=spec["in_specs"],
        out_specs=spec["out_specs"],
        out_shape=spec["out_shape"],
        scratch_shapes=spec["scratch_shapes"],
        compiler_params=pltpu.CompilerParams(
            dimension_semantics=("arbitrary", "arbitrary", "arbitrary")),
    )(bb_p, pri_t)

    match = match3.reshape(PPAD)
    miou = miou3.reshape(PPAD)
    bbt = bb_p.T
    dcx, dcy, dw, dh, cls = _k2_call()(
        match, miou, bbt[0], bbt[1], bbt[2], bbt[3], lab_p,
        pri_t[0], pri_t[1], pri_t[2], pri_t[3])

    reg = jnp.stack([dcx[:P], dcy[:P], dw[:P], dh[:P]], axis=-1)
    return reg, cls[:P]
